# baseline (speedup 1.0000x reference)
import functools

import jax
import jax.numpy as jnp
from jax import lax
from jax.experimental import pallas as pl
from jax.experimental.pallas import tpu as pltpu
from jax.experimental.pallas import tpu_sc as plsc

_CHUNK = 128
_NBUF = 4


@functools.cache
def _build_gather(B, V, D, num_cores, num_subcores):
    nw = num_cores * num_subcores
    assert B % (nw * _CHUNK) == 0
    rows_per_w = B // nw
    n = rows_per_w // _CHUNK
    assert n > 2 * _NBUF and (n - 2 * _NBUF) % _NBUF == 0

    mesh = plsc.VectorSubcoreMesh(core_axis_name="c", subcore_axis_name="s")

    scratch = (
        [pltpu.VMEM((n, _CHUNK), jnp.int32)]
        + [pltpu.VMEM((_CHUNK, 128), jnp.float32) for _ in range(_NBUF)]
        + [pltpu.SemaphoreType.DMA for _ in range(2 * _NBUF)]
    )

    @functools.partial(
        pl.kernel,
        mesh=mesh,
        out_type=jax.ShapeDtypeStruct((B, 128), jnp.float32),
        scratch_types=scratch,
        compiler_params=pltpu.CompilerParams(use_tc_tiling_on_sc=True),
    )
    def gather(idx_hbm, table_hbm, out_hbm, idx_v, *bufs_and_sems):
        bufs = bufs_and_sems[:_NBUF]
        sems_g = bufs_and_sems[_NBUF : 2 * _NBUF]
        sems_s = bufs_and_sems[2 * _NBUF :]

        wid = lax.axis_index("s") * num_cores + lax.axis_index("c")
        base_chunk = wid * n
        base_row = wid * rows_per_w
        pltpu.sync_copy(idx_hbm.at[pl.ds(base_chunk, n)], idx_v)

        def gather_copy(j, b):
            return pltpu.make_async_copy(
                table_hbm.at[idx_v.at[j]],
                bufs[b],
                sems_g[b],
            )

        def scatter_copy(j, b):
            return pltpu.make_async_copy(
                bufs[b],
                out_hbm.at[pl.ds(base_row + j * _CHUNK, _CHUNK)],
                sems_s[b],
            )

        for b in range(_NBUF):
            gather_copy(b, b).start()

        for k in range(_NBUF):
            gather_copy(k, k).wait()
            scatter_copy(k, k).start()
            if k >= 1:
                bp = k - 1
                scatter_copy(k - 1, bp).wait()
                gather_copy(k - 1 + _NBUF, bp).start()

        @pl.loop(_NBUF, n - _NBUF, step=_NBUF)
        def _body(ko):
            for b in range(_NBUF):
                k = ko + b
                gather_copy(k, b).wait()
                scatter_copy(k, b).start()
                bp = (b - 1) % _NBUF
                scatter_copy(k - 1, bp).wait()
                gather_copy(k - 1 + _NBUF, bp).start()

        for k in range(n - _NBUF, n):
            b = k % _NBUF
            gather_copy(k, b).wait()
            scatter_copy(k, b).start()
            bp = (b - 1) % _NBUF
            scatter_copy(k - 1, bp).wait()
            if k - 1 + _NBUF < n:
                gather_copy(k - 1 + _NBUF, bp).start()
        scatter_copy(n - 1, (n - 1) % _NBUF).wait()

    return gather


def kernel(x, table):
    batch, seq = x.shape
    V, D = table.shape
    B = batch * seq
    info = plsc.get_sparse_core_info()
    xf = x.reshape(B // _CHUNK, _CHUNK)
    tpad = jnp.pad(table, ((0, 0), (0, 128 - D)))
    out = _build_gather(B, V, D, info.num_cores, info.num_subcores)(xf, tpad)
    return out[:, :D].reshape(batch, seq, D)


# speedup vs baseline: 1.0902x; 1.0902x over previous
import functools

import jax
import jax.numpy as jnp
from jax import lax
from jax.experimental import pallas as pl
from jax.experimental.pallas import tpu as pltpu
from jax.experimental.pallas import tpu_sc as plsc

_CHUNK = 128
_NBUF = 8


@functools.cache
def _build_gather(B, V, D, num_cores, num_subcores):
    nw = num_cores * num_subcores
    assert B % (nw * _CHUNK) == 0
    rows_per_w = B // nw
    n = rows_per_w // _CHUNK
    assert n > 2 * _NBUF and (n - 2 * _NBUF) % _NBUF == 0

    mesh = plsc.VectorSubcoreMesh(core_axis_name="c", subcore_axis_name="s")

    scratch = (
        [pltpu.VMEM((n, _CHUNK), jnp.int32)]
        + [pltpu.VMEM((_CHUNK, D), jnp.float32) for _ in range(_NBUF)]
        + [pltpu.SemaphoreType.DMA for _ in range(2 * _NBUF)]
    )

    @functools.partial(
        pl.kernel,
        mesh=mesh,
        out_type=jax.ShapeDtypeStruct((B, 128), jnp.float32),
        scratch_types=scratch,
        compiler_params=pltpu.CompilerParams(use_tc_tiling_on_sc=False),
    )
    def gather(idx_hbm, table_hbm, out_hbm, idx_v, *bufs_and_sems):
        bufs = bufs_and_sems[:_NBUF]
        sems_g = bufs_and_sems[_NBUF : 2 * _NBUF]
        sems_s = bufs_and_sems[2 * _NBUF :]

        wid = lax.axis_index("s") * num_cores + lax.axis_index("c")
        base_chunk = wid * n
        base_row = wid * rows_per_w
        pltpu.sync_copy(idx_hbm.at[pl.ds(base_chunk, n)], idx_v)

        def gather_copy(j, b):
            return pltpu.make_async_copy(
                table_hbm.at[idx_v.at[j]],
                bufs[b],
                sems_g[b],
            )

        def scatter_copy(j, b):
            return pltpu.make_async_copy(
                bufs[b],
                out_hbm.at[pl.ds(base_row + j * _CHUNK, _CHUNK), pl.ds(0, D)],
                sems_s[b],
            )

        for b in range(_NBUF):
            gather_copy(b, b).start()

        for k in range(_NBUF):
            gather_copy(k, k).wait()
            scatter_copy(k, k).start()
            if k >= 1:
                bp = k - 1
                scatter_copy(k - 1, bp).wait()
                gather_copy(k - 1 + _NBUF, bp).start()

        @pl.loop(_NBUF, n - _NBUF, step=_NBUF)
        def _body(ko):
            for b in range(_NBUF):
                k = ko + b
                gather_copy(k, b).wait()
                scatter_copy(k, b).start()
                bp = (b - 1) % _NBUF
                scatter_copy(k - 1, bp).wait()
                gather_copy(k - 1 + _NBUF, bp).start()

        for k in range(n - _NBUF, n):
            b = k % _NBUF
            gather_copy(k, b).wait()
            scatter_copy(k, b).start()
            bp = (b - 1) % _NBUF
            scatter_copy(k - 1, bp).wait()
            if k - 1 + _NBUF < n:
                gather_copy(k - 1 + _NBUF, bp).start()
        scatter_copy(n - 1, (n - 1) % _NBUF).wait()

    return gather


def kernel(x, table):
    batch, seq = x.shape
    V, D = table.shape
    B = batch * seq
    info = plsc.get_sparse_core_info()
    xf = x.reshape(B // _CHUNK, _CHUNK)
    out = _build_gather(B, V, D, info.num_cores, info.num_subcores)(xf, table)
    return out[:, :D].reshape(batch, seq, D)
